# baseline (speedup 1.0000x reference)
import functools
import math

import jax
import jax.numpy as jnp
from jax import lax
from jax.experimental import pallas as pl
from jax.experimental.pallas import tpu as pltpu
from jax.experimental.pallas import tpu_sc as plsc

N = 10000
E = 160000
D = 256
H = 512
C = 10
G = 64

N_PAD = 10240
E_PAD = 163840

N_SC = 2
N_TILE = 16
CHUNK = 128
EDGES_PER_TILE = E_PAD // N_TILE
EDGE_ITERS = EDGES_PER_TILE // CHUNK
ROWS_PER_TILE = N_PAD // N_TILE

R = 512
M = N_PAD // R

_INV_BN = 1.0 / math.sqrt(1.0 + 1e-5)


def _leaky(v):
    return jnp.where(v >= 0, v, 0.2 * v)


NBUF = 2
SKEW = NBUF // 2
IBUF = 8
GROUP = 8
OUTER = EDGE_ITERS // GROUP


def _make_sc_agg(n128):
    mesh = plsc.VectorSubcoreMesh(core_axis_name="c", subcore_axis_name="s")

    @functools.partial(
        pl.kernel,
        out_type=jax.ShapeDtypeStruct((n128, N_PAD, 128), jnp.float32),
        mesh=mesh,
        scratch_types=[
            pltpu.VMEM((IBUF, CHUNK), jnp.int32),
            pltpu.VMEM((IBUF, CHUNK), jnp.int32),
        ] + [pltpu.VMEM((CHUNK, 128), jnp.float32)] * NBUF
          + [pltpu.SemaphoreType.DMA] * (2 * NBUF + 2 * IBUF)
          + [pltpu.VMEM_SHARED((N_PAD, 128), jnp.float32)],
    )
    def agg_kernel(tbl, src2, dst2, zeros, out, ibuf, dbuf, *rest):
        rows = rest[:NBUF]
        sem_g = rest[NBUF:2 * NBUF]
        sem_s = rest[2 * NBUF:3 * NBUF]
        sem_i = rest[3 * NBUF:3 * NBUF + IBUF]
        sem_d = rest[3 * NBUF + IBUF:3 * NBUF + 2 * IBUF]
        slab = rest[-1]
        cid = lax.axis_index("c")
        sid = lax.axis_index("s")
        row0 = sid * ROWS_PER_TILE
        erow = sid * EDGE_ITERS

        def ifire(j, t):
            pltpu.async_copy(src2.at[erow + j], ibuf.at[t], sem_i[t])
            pltpu.async_copy(dst2.at[erow + j], dbuf.at[t], sem_d[t])

        def iwait(j, t):
            pltpu.make_async_copy(src2.at[erow + j], ibuf.at[t],
                                  sem_i[t]).wait()

        def dwait(j, t):
            pltpu.make_async_copy(dst2.at[erow + j], dbuf.at[t],
                                  sem_d[t]).wait()

        for p in range(n128 // N_SC):
            chunk = cid * (n128 // N_SC) + p
            pltpu.sync_copy(
                zeros.at[pl.ds(row0, ROWS_PER_TILE)],
                slab.at[pl.ds(row0, ROWS_PER_TILE)],
            )
            plsc.subcore_barrier()

            def gref(t):
                return tbl.at[chunk].at[ibuf.at[t % IBUF]]

            def sref(t):
                return slab.at[dbuf.at[t % IBUF]]

            for t in range(6):
                ifire(t, t)
            for t in range(SKEW):
                iwait(t, t)
                pltpu.async_copy(gref(t), rows[t], sem_g[t])

            def body(i, carry):
                for t in range(GROUP):
                    j = i * GROUP + t
                    b = t % NBUF
                    b2 = (t + SKEW) % NBUF

                    @pl.when(j + SKEW < EDGE_ITERS)
                    def _():
                        @pl.when(j >= SKEW)
                        def _():
                            pltpu.make_async_copy(
                                rows[b2], sref(t - SKEW), sem_s[b2]).wait()
                        iwait(j + SKEW, (t + SKEW) % IBUF)
                        pltpu.async_copy(gref(t + SKEW), rows[b2], sem_g[b2])

                    @pl.when(j + 6 < EDGE_ITERS)
                    def _():
                        ifire(j + 6, (t + 6) % IBUF)

                    pltpu.make_async_copy(gref(t), rows[b], sem_g[b]).wait()
                    dwait(j, t % IBUF)
                    pltpu.async_copy(rows[b], sref(t), sem_s[b], add=True)
                return carry

            lax.fori_loop(0, OUTER, body, 0)
            for jl in range(EDGE_ITERS - NBUF, EDGE_ITERS):
                pltpu.make_async_copy(
                    rows[jl % NBUF], sref(jl), sem_s[jl % NBUF]).wait()
            plsc.subcore_barrier()
            pltpu.sync_copy(
                slab.at[pl.ds(row0, ROWS_PER_TILE)],
                out.at[chunk, pl.ds(row0, ROWS_PER_TILE), :],
            )
            plsc.subcore_barrier()

    return agg_kernel


_sc_agg_cache = {}


def _sc_agg(n_tbl, *args):
    fn = _sc_agg_cache.get(n_tbl)
    if fn is None:
        fn = _make_sc_agg(n_tbl)
        _sc_agg_cache[n_tbl] = fn
    return fn(*args)


def _tc1_body(x2, agg, W1, b1, g1, bb1, sc1, out):
    xb = jnp.concatenate([x2[0], x2[1]], axis=1)
    ab = jnp.concatenate([agg[0], agg[1]], axis=1)
    y = _bdot(sc1[0, 0] * xb + ab, W1[...])
    y = (y + b1[0]) * (g1[0] * _INV_BN) + bb1[0]
    h = _leaky(y)
    for c in range(4):
        out[c] = h[:, c * 128:(c + 1) * 128]


def _tc1(x2, agg1, W1, b1, g1, bb1, sc1):
    row = lambda i: (0, i, 0)
    fix = lambda i: (0, 0)
    return pl.pallas_call(
        _tc1_body,
        grid=(M,),
        in_specs=[
            pl.BlockSpec((2, R, 128), row),
            pl.BlockSpec((2, R, 128), row),
            pl.BlockSpec((D, H), fix),
            pl.BlockSpec((1, H), fix),
            pl.BlockSpec((1, H), fix),
            pl.BlockSpec((1, H), fix),
            pl.BlockSpec((1, 128), fix),
        ],
        out_specs=pl.BlockSpec((4, R, 128), row),
        out_shape=jax.ShapeDtypeStruct((4, N_PAD, 128), jnp.float32),
    )(x2, agg1, W1, b1, g1, bb1, sc1)


def _bdot(a, w):
    return jnp.dot(a, w, preferred_element_type=jnp.float32)


def _tc2_body(h14, agg4, W2, b2, g2, bb2, sc2, Wm, bm, brow3,
              Wc1, bc1, gc, bbc, Wc2, bc2,
              Ws1, bs1, gs, bbs, Ws2, bs2,
              Wt1, bt1, gt, bbt, Wt2, bt2,
              h3_out, oc, osv, ost, S_acc):
    i = pl.program_id(0)
    hb = jnp.concatenate([h14[c] for c in range(4)], axis=1)
    ab = jnp.concatenate([agg4[c] for c in range(4)], axis=1)
    t = _bdot(sc2[0, 0] * hb + ab, W2[...])
    t = _leaky((t + b2[0]) * (g2[0] * _INV_BN) + bb2[0])
    h3 = _leaky(_bdot(t, Wm[...]) + bm[0])
    h3_out[...] = h3

    brow = brow3[0]
    oh = (lax.broadcasted_iota(jnp.int32, (G, R), 0) == brow).astype(jnp.float32)
    xcat = jnp.concatenate([h3, jnp.ones((R, 128), jnp.float32)], axis=1)
    contrib = jnp.dot(oh, xcat, preferred_element_type=jnp.float32)

    @pl.when(i == 0)
    def _():
        S_acc[...] = jnp.zeros_like(S_acc)

    S_acc[...] += contrib

    @pl.when(i == M - 1)
    def _():
        S = S_acc[...]
        sums = S[:, :H]
        cnt = S[:, H:H + 1]
        pooled = sums / jnp.maximum(cnt, 1.0)

        def head(Wa, ba, g, bb, Wb, bo):
            z = jnp.dot(pooled, Wa[...],
                        preferred_element_type=jnp.float32) + ba[0]
            z = jnp.maximum(z, 0.0)
            z = z * (g[0] * _INV_BN) + bb[0]
            return jnp.dot(z, Wb[...],
                           preferred_element_type=jnp.float32) + bo[0]

        oc[...] = head(Wc1, bc1, gc, bbc, Wc2, bc2)
        osv[...] = head(Ws1, bs1, gs, bbs, Ws2, bs2)
        ost[...] = head(Wt1, bt1, gt, bbt, Wt2, bt2)


def _tc2(h14, agg2, W2, b2, g2, bb2, sc2, Wm, bm, brow3,
         Wc1, bc1, gc, bbc, Wc2, bc2, Ws1, bs1, gs, bbs, Ws2, bs2,
         Wt1, bt1, gt, bbt, Wt2, bt2):
    row = lambda i: (0, i, 0)
    fix = lambda i: (0, 0)
    wspec = lambda a: pl.BlockSpec(a.shape, fix)
    heads_in = (Wc1, bc1, gc, bbc, Wc2, bc2, Ws1, bs1, gs, bbs, Ws2, bs2,
                Wt1, bt1, gt, bbt, Wt2, bt2)
    return pl.pallas_call(
        _tc2_body,
        grid=(M,),
        in_specs=[
            pl.BlockSpec((4, R, 128), row),
            pl.BlockSpec((4, R, 128), row),
            pl.BlockSpec((H, H), fix),
            pl.BlockSpec((1, H), fix),
            pl.BlockSpec((1, H), fix),
            pl.BlockSpec((1, H), fix),
            pl.BlockSpec((1, 128), fix),
            pl.BlockSpec((H, H), fix),
            pl.BlockSpec((1, H), fix),
            pl.BlockSpec((1, 1, R), lambda i: (i, 0, 0)),
        ] + [wspec(a) for a in heads_in],
        out_specs=[
            pl.BlockSpec((R, H), lambda i: (i, 0)),
            pl.BlockSpec((G, C), fix),
            pl.BlockSpec((G, 1), fix),
            pl.BlockSpec((G, 2), fix),
        ],
        out_shape=[
            jax.ShapeDtypeStruct((N_PAD, H), jnp.float32),
            jax.ShapeDtypeStruct((G, C), jnp.float32),
            jax.ShapeDtypeStruct((G, 1), jnp.float32),
            jax.ShapeDtypeStruct((G, 2), jnp.float32),
        ],
        scratch_shapes=[pltpu.VMEM((G, H + 128), jnp.float32)],
    )(h14, agg2, W2, b2, g2, bb2, sc2, Wm, bm, brow3, *heads_in)


def kernel(x, edge_index, batch, eps1, W1, b1, bn1_g, bn1_b, eps2, W2, b2,
           bn2_g, bn2_b, Wm, bm, Wc1, bc1, bnc_g, bnc_b, Wc2, bc2, Ws1, bs1,
           bns_g, bns_b, Ws2, bs2, Wt1, bt1, bnt_g, bnt_b, Wt2, bt2):
    src = edge_index[0]
    dst = edge_index[1]
    pad_e = jnp.full((E_PAD - E,), N, jnp.int32)
    src_p = jnp.concatenate([src, pad_e]).reshape(E_PAD // CHUNK, CHUNK)
    dst_p = jnp.concatenate([dst, pad_e]).reshape(E_PAD // CHUNK, CHUNK)

    xp = jnp.zeros((N_PAD, D), jnp.float32).at[:N].set(x)
    x2 = xp.reshape(N_PAD, 2, 128).transpose(1, 0, 2)
    zeros_tbl = jnp.zeros((N_PAD, 128), jnp.float32)

    rowv = lambda a: a.reshape(1, -1)
    sc1 = jnp.full((1, 128), 2.0 + eps1, jnp.float32)
    sc2 = jnp.full((1, 128), 2.0 + eps2, jnp.float32)

    agg1 = _sc_agg(2, x2, src_p, dst_p, zeros_tbl)
    h14 = _tc1(x2, agg1, W1, rowv(b1), rowv(bn1_g), rowv(bn1_b), sc1)
    agg2 = _sc_agg(4, h14, src_p, dst_p, zeros_tbl)

    batch_p = jnp.concatenate([batch, jnp.full((N_PAD - N,), G, jnp.int32)])
    brow3 = batch_p.reshape(M, 1, R)

    h3, class_out, surv_out, status_out = _tc2(
        h14, agg2, W2, rowv(b2), rowv(bn2_g), rowv(bn2_b), sc2,
        Wm, rowv(bm), brow3,
        Wc1, rowv(bc1), rowv(bnc_g), rowv(bnc_b), Wc2, rowv(bc2),
        Ws1, rowv(bs1), rowv(bns_g), rowv(bns_b), Ws2, rowv(bs2),
        Wt1, rowv(bt1), rowv(bnt_g), rowv(bnt_b), Wt2, rowv(bt2))

    return (h3[:N], class_out, surv_out, status_out)


# speedup vs baseline: 2.4145x; 2.4145x over previous
import functools
import math

import jax
import jax.numpy as jnp
from jax import lax
from jax.experimental import pallas as pl
from jax.experimental.pallas import tpu as pltpu
from jax.experimental.pallas import tpu_sc as plsc

N = 10000
E = 160000
D = 256
H = 512
C = 10
G = 64

N_PAD = 10240
E_PAD = 163840

N_SC = 2
N_TILE = 16
CHUNK = 128
EDGES_PER_TILE = E_PAD // N_TILE
EDGE_ITERS = EDGES_PER_TILE // CHUNK
ROWS_PER_TILE = N_PAD // N_TILE

R = 512
M = N_PAD // R

_INV_BN = 1.0 / math.sqrt(1.0 + 1e-5)


def _leaky(v):
    return jnp.where(v >= 0, v, 0.2 * v)


NBUF = 2
SKEW = NBUF // 2
IBUF = 8
GROUP = 8
OUTER = EDGE_ITERS // GROUP


def _make_sc_agg(n128):
    mesh = plsc.VectorSubcoreMesh(core_axis_name="c", subcore_axis_name="s")

    @functools.partial(
        pl.kernel,
        out_type=jax.ShapeDtypeStruct((n128, N_PAD, 128), jnp.float32),
        mesh=mesh,
        scratch_types=[
            pltpu.VMEM((IBUF, CHUNK), jnp.int32),
            pltpu.VMEM((IBUF, CHUNK), jnp.int32),
        ] + [pltpu.VMEM((CHUNK, 128), jnp.float32)] * NBUF
          + [pltpu.SemaphoreType.DMA] * (2 * NBUF + 2 * IBUF)
          + [pltpu.VMEM_SHARED((N_PAD, 128), jnp.float32)],
    )
    def agg_kernel(tbl, src2, dst2, zeros, out, ibuf, dbuf, *rest):
        rows = rest[:NBUF]
        sem_g = rest[NBUF:2 * NBUF]
        sem_s = rest[2 * NBUF:3 * NBUF]
        sem_i = rest[3 * NBUF:3 * NBUF + IBUF]
        sem_d = rest[3 * NBUF + IBUF:3 * NBUF + 2 * IBUF]
        slab = rest[-1]
        cid = lax.axis_index("c")
        sid = lax.axis_index("s")
        row0 = sid * ROWS_PER_TILE
        erow = sid * EDGE_ITERS

        def ifire(j, t):
            pltpu.async_copy(src2.at[erow + j], ibuf.at[t], sem_i[t])
            pltpu.async_copy(dst2.at[erow + j], dbuf.at[t], sem_d[t])

        def iwait(j, t):
            pltpu.make_async_copy(src2.at[erow + j], ibuf.at[t],
                                  sem_i[t]).wait()

        def dwait(j, t):
            pltpu.make_async_copy(dst2.at[erow + j], dbuf.at[t],
                                  sem_d[t]).wait()

        for p in range(n128 // N_SC):
            chunk = cid * (n128 // N_SC) + p
            pltpu.sync_copy(
                zeros.at[pl.ds(row0, ROWS_PER_TILE)],
                slab.at[pl.ds(row0, ROWS_PER_TILE)],
            )
            plsc.subcore_barrier()

            def gref(t):
                return tbl.at[chunk].at[ibuf.at[t % IBUF]]

            def sref(t):
                return slab.at[dbuf.at[t % IBUF]]

            for t in range(6):
                ifire(t, t)
            for t in range(SKEW):
                iwait(t, t)
                pltpu.async_copy(gref(t), rows[t], sem_g[t])

            def body(i, carry):
                for t in range(GROUP):
                    j = i * GROUP + t
                    b = t % NBUF
                    b2 = (t + SKEW) % NBUF

                    @pl.when(j + SKEW < EDGE_ITERS)
                    def _():
                        @pl.when(j >= SKEW)
                        def _():
                            pltpu.make_async_copy(
                                rows[b2], sref(t - SKEW), sem_s[b2]).wait()
                        iwait(j + SKEW, (t + SKEW) % IBUF)
                        pltpu.async_copy(gref(t + SKEW), rows[b2], sem_g[b2])

                    @pl.when(j + 6 < EDGE_ITERS)
                    def _():
                        ifire(j + 6, (t + 6) % IBUF)

                    pltpu.make_async_copy(gref(t), rows[b], sem_g[b]).wait()
                    dwait(j, t % IBUF)
                    pltpu.async_copy(rows[b], sref(t), sem_s[b], add=True)
                return carry

            lax.fori_loop(0, OUTER, body, 0)
            for jl in range(EDGE_ITERS - NBUF, EDGE_ITERS):
                pltpu.make_async_copy(
                    rows[jl % NBUF], sref(jl), sem_s[jl % NBUF]).wait()
            plsc.subcore_barrier()
            pltpu.sync_copy(
                slab.at[pl.ds(row0, ROWS_PER_TILE)],
                out.at[chunk, pl.ds(row0, ROWS_PER_TILE), :],
            )
            plsc.subcore_barrier()

    return agg_kernel


_sc_agg_cache = {}


def _sc_agg(n_tbl, *args):
    fn = _sc_agg_cache.get(n_tbl)
    if fn is None:
        fn = _make_sc_agg(n_tbl)
        _sc_agg_cache[n_tbl] = fn
    return fn(*args)


def _tc1_body(x2, agg, W1, b1, g1, bb1, sc1, out):
    xb = jnp.concatenate([x2[0], x2[1]], axis=1)
    ab = jnp.concatenate([agg[0], agg[1]], axis=1)
    y = jnp.dot(sc1[0, 0] * xb + ab, W1[...], preferred_element_type=jnp.float32)
    y = (y + b1[0]) * (g1[0] * _INV_BN) + bb1[0]
    h = _leaky(y)
    for c in range(4):
        out[c] = h[:, c * 128:(c + 1) * 128]


def _tc1(x2, agg1, W1, b1, g1, bb1, sc1):
    row = lambda i: (0, i, 0)
    fix = lambda i: (0, 0)
    return pl.pallas_call(
        _tc1_body,
        grid=(M,),
        in_specs=[
            pl.BlockSpec((2, R, 128), row),
            pl.BlockSpec((2, R, 128), row),
            pl.BlockSpec((D, H), fix),
            pl.BlockSpec((1, H), fix),
            pl.BlockSpec((1, H), fix),
            pl.BlockSpec((1, H), fix),
            pl.BlockSpec((1, 128), fix),
        ],
        out_specs=pl.BlockSpec((4, R, 128), row),
        out_shape=jax.ShapeDtypeStruct((4, N_PAD, 128), jnp.float32),
    )(x2, agg1, W1, b1, g1, bb1, sc1)


def _tc2_body(h14, agg4, W2, b2, g2, bb2, sc2, Wm, bm, brow3, h3_out, S_out):
    i = pl.program_id(0)
    hb = jnp.concatenate([h14[c] for c in range(4)], axis=1)
    ab = jnp.concatenate([agg4[c] for c in range(4)], axis=1)
    t = jnp.dot(sc2[0, 0] * hb + ab, W2[...], preferred_element_type=jnp.float32)
    t = _leaky((t + b2[0]) * (g2[0] * _INV_BN) + bb2[0])
    h3 = _leaky(jnp.dot(t, Wm[...], preferred_element_type=jnp.float32) + bm[0])
    h3_out[...] = h3

    brow = brow3[0]
    oh = (lax.broadcasted_iota(jnp.int32, (G, R), 0) == brow).astype(jnp.float32)
    xcat = jnp.concatenate([h3, jnp.ones((R, 128), jnp.float32)], axis=1)
    contrib = jnp.dot(oh, xcat, preferred_element_type=jnp.float32)

    @pl.when(i == 0)
    def _():
        S_out[...] = jnp.zeros_like(S_out)

    S_out[...] += contrib


def _tc2(h14, agg2, W2, b2, g2, bb2, sc2, Wm, bm, brow3):
    row = lambda i: (0, i, 0)
    fix = lambda i: (0, 0)
    return pl.pallas_call(
        _tc2_body,
        grid=(M,),
        in_specs=[
            pl.BlockSpec((4, R, 128), row),
            pl.BlockSpec((4, R, 128), row),
            pl.BlockSpec((H, H), fix),
            pl.BlockSpec((1, H), fix),
            pl.BlockSpec((1, H), fix),
            pl.BlockSpec((1, H), fix),
            pl.BlockSpec((1, 128), fix),
            pl.BlockSpec((H, H), fix),
            pl.BlockSpec((1, H), fix),
            pl.BlockSpec((1, 1, R), lambda i: (i, 0, 0)),
        ],
        out_specs=[
            pl.BlockSpec((R, H), lambda i: (i, 0)),
            pl.BlockSpec((G, H + 128), fix),
        ],
        out_shape=[
            jax.ShapeDtypeStruct((N_PAD, H), jnp.float32),
            jax.ShapeDtypeStruct((G, H + 128), jnp.float32),
        ],
    )(h14, agg2, W2, b2, g2, bb2, sc2, Wm, bm, brow3)


def _tc3_body(S, Wc1, bc1, gc, bbc, Wc2, bc2,
              Ws1, bs1, gs, bbs, Ws2, bs2,
              Wt1, bt1, gt, bbt, Wt2, bt2, oc, osv, ost):
    sums = S[:, :H]
    cnt = S[:, H:H + 1]
    pooled = sums / jnp.maximum(cnt, 1.0)

    def head(Wa, ba, g, bb, Wb, bo):
        z = jnp.dot(pooled, Wa[...], preferred_element_type=jnp.float32) + ba[0]
        z = jnp.maximum(z, 0.0)
        z = z * (g[0] * _INV_BN) + bb[0]
        return jnp.dot(z, Wb[...], preferred_element_type=jnp.float32) + bo[0]

    oc[...] = head(Wc1, bc1, gc, bbc, Wc2, bc2)
    osv[...] = head(Ws1, bs1, gs, bbs, Ws2, bs2)
    ost[...] = head(Wt1, bt1, gt, bbt, Wt2, bt2)


def _tc3(S, Wc1, bc1, gc, bbc, Wc2, bc2, Ws1, bs1, gs, bbs, Ws2, bs2,
         Wt1, bt1, gt, bbt, Wt2, bt2):
    args = (S, Wc1, bc1, gc, bbc, Wc2, bc2, Ws1, bs1, gs, bbs, Ws2, bs2,
            Wt1, bt1, gt, bbt, Wt2, bt2)
    return pl.pallas_call(
        _tc3_body,
        in_specs=[pl.BlockSpec(a.shape, lambda: (0,) * a.ndim) for a in args],
        out_specs=[
            pl.BlockSpec((G, C), lambda: (0, 0)),
            pl.BlockSpec((G, 1), lambda: (0, 0)),
            pl.BlockSpec((G, 2), lambda: (0, 0)),
        ],
        out_shape=[
            jax.ShapeDtypeStruct((G, C), jnp.float32),
            jax.ShapeDtypeStruct((G, 1), jnp.float32),
            jax.ShapeDtypeStruct((G, 2), jnp.float32),
        ],
    )(*args)


def kernel(x, edge_index, batch, eps1, W1, b1, bn1_g, bn1_b, eps2, W2, b2,
           bn2_g, bn2_b, Wm, bm, Wc1, bc1, bnc_g, bnc_b, Wc2, bc2, Ws1, bs1,
           bns_g, bns_b, Ws2, bs2, Wt1, bt1, bnt_g, bnt_b, Wt2, bt2):
    src = edge_index[0]
    dst = edge_index[1]
    pad_e = N + jnp.arange(E_PAD - E, dtype=jnp.int32) % (N_PAD - N)
    src_p = jnp.concatenate([src, pad_e]).reshape(E_PAD // CHUNK, CHUNK)
    dst_p = jnp.concatenate([dst, pad_e]).reshape(E_PAD // CHUNK, CHUNK)

    xp = jnp.zeros((N_PAD, D), jnp.float32).at[:N].set(x)
    x2 = xp.reshape(N_PAD, 2, 128).transpose(1, 0, 2)
    zeros_tbl = jnp.zeros((N_PAD, 128), jnp.float32)

    rowv = lambda a: a.reshape(1, -1)
    sc1 = jnp.full((1, 128), 2.0 + eps1, jnp.float32)
    sc2 = jnp.full((1, 128), 2.0 + eps2, jnp.float32)

    agg1 = _sc_agg(2, x2, src_p, dst_p, zeros_tbl)
    h14 = _tc1(x2, agg1, W1, rowv(b1), rowv(bn1_g), rowv(bn1_b), sc1)
    agg2 = _sc_agg(4, h14, src_p, dst_p, zeros_tbl)

    batch_p = jnp.concatenate([batch, jnp.full((N_PAD - N,), G, jnp.int32)])
    brow3 = batch_p.reshape(M, 1, R)

    h3, S = _tc2(h14, agg2, W2, rowv(b2), rowv(bn2_g), rowv(bn2_b), sc2,
                 Wm, rowv(bm), brow3)

    class_out, surv_out, status_out = _tc3(
        S, Wc1, rowv(bc1), rowv(bnc_g), rowv(bnc_b), Wc2, rowv(bc2),
        Ws1, rowv(bs1), rowv(bns_g), rowv(bns_b), Ws2, rowv(bs2),
        Wt1, rowv(bt1), rowv(bnt_g), rowv(bnt_b), Wt2, rowv(bt2))

    return (h3[:N], class_out, surv_out, status_out)


# speedup vs baseline: 2.4841x; 1.0288x over previous
import functools
import math

import jax
import jax.numpy as jnp
from jax import lax
from jax.experimental import pallas as pl
from jax.experimental.pallas import tpu as pltpu
from jax.experimental.pallas import tpu_sc as plsc

N = 10000
E = 160000
D = 256
H = 512
C = 10
G = 64

N_PAD = 10240
E_PAD = 163840

N_SC = 2
N_TILE = 16
CHUNK = 128
EDGES_PER_TILE = E_PAD // N_TILE
EDGE_ITERS = EDGES_PER_TILE // CHUNK
ROWS_PER_TILE = N_PAD // N_TILE

R = 512
M = N_PAD // R

_INV_BN = 1.0 / math.sqrt(1.0 + 1e-5)


def _leaky(v):
    return jnp.where(v >= 0, v, 0.2 * v)


NBUF = 2
SKEW = NBUF // 2
IBUF = 8
GROUP = 8
OUTER = EDGE_ITERS // GROUP


def _make_sc_agg(n128):
    mesh = plsc.VectorSubcoreMesh(core_axis_name="c", subcore_axis_name="s")

    @functools.partial(
        pl.kernel,
        out_type=jax.ShapeDtypeStruct((n128, N_PAD, 128), jnp.float32),
        mesh=mesh,
        scratch_types=[
            pltpu.VMEM((IBUF, CHUNK), jnp.int32),
            pltpu.VMEM((IBUF, CHUNK), jnp.int32),
        ] + [pltpu.VMEM((CHUNK, 128), jnp.float32)] * NBUF
          + [pltpu.SemaphoreType.DMA] * (2 * NBUF + 2 * IBUF)
          + [pltpu.VMEM_SHARED((N_PAD, 128), jnp.float32)],
    )
    def agg_kernel(tbl, src2, dst2, zeros, out, ibuf, dbuf, *rest):
        rows = rest[:NBUF]
        sem_g = rest[NBUF:2 * NBUF]
        sem_s = rest[2 * NBUF:3 * NBUF]
        sem_i = rest[3 * NBUF:3 * NBUF + IBUF]
        sem_d = rest[3 * NBUF + IBUF:3 * NBUF + 2 * IBUF]
        slab = rest[-1]
        cid = lax.axis_index("c")
        sid = lax.axis_index("s")
        row0 = sid * ROWS_PER_TILE
        erow = sid * EDGE_ITERS

        def ifire(j, t):
            pltpu.async_copy(src2.at[erow + j], ibuf.at[t], sem_i[t])
            pltpu.async_copy(dst2.at[erow + j], dbuf.at[t], sem_d[t])

        def iwait(j, t):
            pltpu.make_async_copy(src2.at[erow + j], ibuf.at[t],
                                  sem_i[t]).wait()

        def dwait(j, t):
            pltpu.make_async_copy(dst2.at[erow + j], dbuf.at[t],
                                  sem_d[t]).wait()

        for p in range(n128 // N_SC):
            chunk = cid * (n128 // N_SC) + p
            pltpu.sync_copy(
                zeros.at[pl.ds(row0, ROWS_PER_TILE)],
                slab.at[pl.ds(row0, ROWS_PER_TILE)],
            )
            plsc.subcore_barrier()

            def gref(t):
                return tbl.at[chunk].at[ibuf.at[t % IBUF]]

            def sref(t):
                return slab.at[dbuf.at[t % IBUF]]

            for t in range(6):
                ifire(t, t)
            for t in range(SKEW):
                iwait(t, t)
                pltpu.async_copy(gref(t), rows[t], sem_g[t])

            def body(i, carry):
                for t in range(GROUP):
                    j = i * GROUP + t
                    b = t % NBUF
                    b2 = (t + SKEW) % NBUF

                    @pl.when(j + SKEW < EDGE_ITERS)
                    def _():
                        @pl.when(j >= SKEW)
                        def _():
                            pltpu.make_async_copy(
                                rows[b2], sref(t - SKEW), sem_s[b2]).wait()
                        iwait(j + SKEW, (t + SKEW) % IBUF)
                        pltpu.async_copy(gref(t + SKEW), rows[b2], sem_g[b2])

                    @pl.when(j + 6 < EDGE_ITERS)
                    def _():
                        ifire(j + 6, (t + 6) % IBUF)

                    pltpu.make_async_copy(gref(t), rows[b], sem_g[b]).wait()
                    dwait(j, t % IBUF)
                    pltpu.async_copy(rows[b], sref(t), sem_s[b], add=True)
                return carry

            lax.fori_loop(0, OUTER, body, 0)
            for jl in range(EDGE_ITERS - NBUF, EDGE_ITERS):
                pltpu.make_async_copy(
                    rows[jl % NBUF], sref(jl), sem_s[jl % NBUF]).wait()
            plsc.subcore_barrier()
            pltpu.sync_copy(
                slab.at[pl.ds(row0, ROWS_PER_TILE)],
                out.at[chunk, pl.ds(row0, ROWS_PER_TILE), :],
            )
            plsc.subcore_barrier()

    return agg_kernel


_sc_agg_cache = {}


def _sc_agg(n_tbl, *args):
    fn = _sc_agg_cache.get(n_tbl)
    if fn is None:
        fn = _make_sc_agg(n_tbl)
        _sc_agg_cache[n_tbl] = fn
    return fn(*args)


def _tc1_body(x2, agg, W1, b1, g1, bb1, sc1, out):
    xb = jnp.concatenate([x2[0], x2[1]], axis=1)
    ab = jnp.concatenate([agg[0], agg[1]], axis=1)
    y = jnp.dot(sc1[0, 0] * xb + ab, W1[...], preferred_element_type=jnp.float32)
    y = (y + b1[0]) * (g1[0] * _INV_BN) + bb1[0]
    h = _leaky(y)
    for c in range(4):
        out[c] = h[:, c * 128:(c + 1) * 128]


def _tc1(x2, agg1, W1, b1, g1, bb1, sc1):
    row = lambda i: (0, i, 0)
    fix = lambda i: (0, 0)
    return pl.pallas_call(
        _tc1_body,
        grid=(M,),
        in_specs=[
            pl.BlockSpec((2, R, 128), row),
            pl.BlockSpec((2, R, 128), row),
            pl.BlockSpec((D, H), fix),
            pl.BlockSpec((1, H), fix),
            pl.BlockSpec((1, H), fix),
            pl.BlockSpec((1, H), fix),
            pl.BlockSpec((1, 128), fix),
        ],
        out_specs=pl.BlockSpec((4, R, 128), row),
        out_shape=jax.ShapeDtypeStruct((4, N_PAD, 128), jnp.float32),
    )(x2, agg1, W1, b1, g1, bb1, sc1)


def _tc2_body(h14, agg4, W2, b2, g2, bb2, sc2, Wm, bm, brow3, h3_out, S_out):
    i = pl.program_id(0)
    hb = jnp.concatenate([h14[c] for c in range(4)], axis=1)
    ab = jnp.concatenate([agg4[c] for c in range(4)], axis=1)
    t = jnp.dot(sc2[0, 0] * hb + ab, W2[...], preferred_element_type=jnp.float32)
    t = _leaky((t + b2[0]) * (g2[0] * _INV_BN) + bb2[0])
    h3 = _leaky(jnp.dot(t, Wm[...], preferred_element_type=jnp.float32) + bm[0])
    h3_out[...] = h3

    brow = brow3[0]
    oh = (lax.broadcasted_iota(jnp.int32, (G, R), 0) == brow).astype(jnp.float32)
    xcat = jnp.concatenate([h3, jnp.ones((R, 128), jnp.float32)], axis=1)
    contrib = jnp.dot(oh, xcat, preferred_element_type=jnp.float32)

    @pl.when(i == 0)
    def _():
        S_out[...] = jnp.zeros_like(S_out)

    S_out[...] += contrib


def _tc2(h14, agg2, W2, b2, g2, bb2, sc2, Wm, bm, brow3):
    row = lambda i: (0, i, 0)
    fix = lambda i: (0, 0)
    return pl.pallas_call(
        _tc2_body,
        grid=(M,),
        in_specs=[
            pl.BlockSpec((4, R, 128), row),
            pl.BlockSpec((4, R, 128), row),
            pl.BlockSpec((H, H), fix),
            pl.BlockSpec((1, H), fix),
            pl.BlockSpec((1, H), fix),
            pl.BlockSpec((1, H), fix),
            pl.BlockSpec((1, 128), fix),
            pl.BlockSpec((H, H), fix),
            pl.BlockSpec((1, H), fix),
            pl.BlockSpec((1, 1, R), lambda i: (i, 0, 0)),
        ],
        out_specs=[
            pl.BlockSpec((R, H), lambda i: (i, 0)),
            pl.BlockSpec((G, H + 128), fix),
        ],
        out_shape=[
            jax.ShapeDtypeStruct((N, H), jnp.float32),
            jax.ShapeDtypeStruct((G, H + 128), jnp.float32),
        ],
    )(h14, agg2, W2, b2, g2, bb2, sc2, Wm, bm, brow3)


def _tc3_body(S, Wc1, bc1, gc, bbc, Wc2, bc2,
              Ws1, bs1, gs, bbs, Ws2, bs2,
              Wt1, bt1, gt, bbt, Wt2, bt2, oc, osv, ost):
    sums = S[:, :H]
    cnt = S[:, H:H + 1]
    pooled = sums / jnp.maximum(cnt, 1.0)

    def head(Wa, ba, g, bb, Wb, bo):
        z = jnp.dot(pooled, Wa[...], preferred_element_type=jnp.float32) + ba[0]
        z = jnp.maximum(z, 0.0)
        z = z * (g[0] * _INV_BN) + bb[0]
        return jnp.dot(z, Wb[...], preferred_element_type=jnp.float32) + bo[0]

    oc[...] = head(Wc1, bc1, gc, bbc, Wc2, bc2)
    osv[...] = head(Ws1, bs1, gs, bbs, Ws2, bs2)
    ost[...] = head(Wt1, bt1, gt, bbt, Wt2, bt2)


def _tc3(S, Wc1, bc1, gc, bbc, Wc2, bc2, Ws1, bs1, gs, bbs, Ws2, bs2,
         Wt1, bt1, gt, bbt, Wt2, bt2):
    args = (S, Wc1, bc1, gc, bbc, Wc2, bc2, Ws1, bs1, gs, bbs, Ws2, bs2,
            Wt1, bt1, gt, bbt, Wt2, bt2)
    return pl.pallas_call(
        _tc3_body,
        in_specs=[pl.BlockSpec(a.shape, lambda: (0,) * a.ndim) for a in args],
        out_specs=[
            pl.BlockSpec((G, C), lambda: (0, 0)),
            pl.BlockSpec((G, 1), lambda: (0, 0)),
            pl.BlockSpec((G, 2), lambda: (0, 0)),
        ],
        out_shape=[
            jax.ShapeDtypeStruct((G, C), jnp.float32),
            jax.ShapeDtypeStruct((G, 1), jnp.float32),
            jax.ShapeDtypeStruct((G, 2), jnp.float32),
        ],
    )(*args)


def kernel(x, edge_index, batch, eps1, W1, b1, bn1_g, bn1_b, eps2, W2, b2,
           bn2_g, bn2_b, Wm, bm, Wc1, bc1, bnc_g, bnc_b, Wc2, bc2, Ws1, bs1,
           bns_g, bns_b, Ws2, bs2, Wt1, bt1, bnt_g, bnt_b, Wt2, bt2):
    src = edge_index[0]
    dst = edge_index[1]
    pad_e = N + jnp.arange(E_PAD - E, dtype=jnp.int32) % (N_PAD - N)
    src_p = jnp.concatenate([src, pad_e]).reshape(E_PAD // CHUNK, CHUNK)
    dst_p = jnp.concatenate([dst, pad_e]).reshape(E_PAD // CHUNK, CHUNK)

    xp = jnp.zeros((N_PAD, D), jnp.float32).at[:N].set(x)
    x2 = xp.reshape(N_PAD, 2, 128).transpose(1, 0, 2)
    zeros_tbl = jnp.zeros((N_PAD, 128), jnp.float32)

    rowv = lambda a: a.reshape(1, -1)
    sc1 = jnp.full((1, 128), 2.0 + eps1, jnp.float32)
    sc2 = jnp.full((1, 128), 2.0 + eps2, jnp.float32)

    agg1 = _sc_agg(2, x2, src_p, dst_p, zeros_tbl)
    h14 = _tc1(x2, agg1, W1, rowv(b1), rowv(bn1_g), rowv(bn1_b), sc1)
    agg2 = _sc_agg(4, h14, src_p, dst_p, zeros_tbl)

    batch_p = jnp.concatenate([batch, jnp.full((N_PAD - N,), G, jnp.int32)])
    brow3 = batch_p.reshape(M, 1, R)

    h3, S = _tc2(h14, agg2, W2, rowv(b2), rowv(bn2_g), rowv(bn2_b), sc2,
                 Wm, rowv(bm), brow3)

    class_out, surv_out, status_out = _tc3(
        S, Wc1, rowv(bc1), rowv(bnc_g), rowv(bnc_b), Wc2, rowv(bc2),
        Ws1, rowv(bs1), rowv(bns_g), rowv(bns_b), Ws2, rowv(bs2),
        Wt1, rowv(bt1), rowv(bnt_g), rowv(bnt_b), Wt2, rowv(bt2))

    return (h3, class_out, surv_out, status_out)
